# baseline (speedup 1.0000x reference)
import functools

import jax
import jax.numpy as jnp
from jax.experimental import pallas as pl
from jax.experimental.pallas import tpu as pltpu

B = 8192
H = 2048
E = 16
C = 4
TOPK = 2
P = 8
TEMP = 0.1
P2 = 2 * P

MR = 256
MS = 256
ME = 512


def _router_body(mm_ref, qf_ref, g_ref, b_ref, qpw_ref, qpb_ref, pbar_ref,
                 xbf_ref, w_ref):
    mm = mm_ref[...]
    mu = jnp.mean(mm, axis=-1, keepdims=True)
    var = jnp.mean((mm - mu) ** 2, axis=-1, keepdims=True)
    xn = (mm - mu) * jax.lax.rsqrt(var + 1e-5) * g_ref[...] + b_ref[...]
    ri = (jnp.dot(xn, qpw_ref[:H, :], preferred_element_type=jnp.float32)
          + jnp.dot(qf_ref[...], qpw_ref[H:, :],
                    preferred_element_type=jnp.float32)
          + qpb_ref[...])
    nrm = jnp.sqrt(jnp.sum(ri * ri, axis=-1, keepdims=True))
    ri = ri / jnp.maximum(nrm, 1e-12)
    scores = jnp.dot(ri, pbar_ref[...], preferred_element_type=jnp.float32)
    logits = scores * (1.0 / TEMP)
    m = jnp.max(logits, axis=-1, keepdims=True)
    p = jnp.exp(logits - m)
    probs = p / jnp.sum(p, axis=-1, keepdims=True)
    cols = jax.lax.broadcasted_iota(jnp.int32, probs.shape, 1)
    p1 = jnp.max(probs, axis=-1, keepdims=True)
    i1 = jnp.min(jnp.where(probs == p1, cols, E), axis=-1, keepdims=True)
    probs2 = jnp.where(cols == i1, -1.0, probs)
    p2 = jnp.max(probs2, axis=-1, keepdims=True)
    i2 = jnp.min(jnp.where(probs2 == p2, cols, E), axis=-1, keepdims=True)
    denom = p1 + p2 + 1e-6
    w1 = p1 / denom
    w2 = p2 / denom
    w_ref[...] = jnp.where(cols == i1, w1, 0.0) + jnp.where(cols == i2, w2, 0.0)
    xbf_ref[...] = xn.astype(jnp.bfloat16)


def _shared_body(x_ref, w1_ref, b1_ref, w2_ref, b2_ref, out_ref):
    h = jnp.dot(x_ref[...], w1_ref[...], preferred_element_type=jnp.float32)
    h = jnp.maximum(h + b1_ref[...], 0.0).astype(jnp.bfloat16)
    out_ref[...] = (jnp.dot(h, w2_ref[...], preferred_element_type=jnp.float32)
                    + b2_ref[...])


def _experts_body(x_ref, w1_ref, b1_ref, w2_ref, b2_ref, out_ref):
    b = pl.program_id(1)
    x = x_ref[pl.ds(b * ME, ME), :]
    h = jnp.dot(x, w1_ref[0], preferred_element_type=jnp.float32)
    h = jnp.maximum(h + b1_ref[0], 0.0).astype(jnp.bfloat16)
    out_ref[0] = (jnp.dot(h, w2_ref[0], preferred_element_type=jnp.float32)
                  + b2_ref[0])


def _combine_body(ao_ref, w_ref, sh_ref, gw_ref, gb_ref, out_ref):
    w = w_ref[...]
    acc = jnp.zeros((ME, P2), dtype=jnp.float32)
    for e in range(E):
        acc = acc + w[:, e][:, None] * ao_ref[e]
    z = (jnp.dot(acc, gw_ref[:P2, :], preferred_element_type=jnp.float32)
         + jnp.dot(sh_ref[...], gw_ref[P2:, :],
                   preferred_element_type=jnp.float32)
         + gb_ref[...])
    out_ref[...] = jax.nn.sigmoid(z)


def kernel(multimodal_feat, query_feat, ln_gamma, ln_beta, pilot, qp_W, qp_b,
           exp_W1, exp_b1, exp_W2, exp_b2, sh_W1, sh_b1, sh_W2, sh_b2,
           gate_W, gate_b):
    f32 = jnp.float32
    pn = pilot / jnp.clip(
        jnp.linalg.norm(pilot, axis=-1, keepdims=True), 1e-12)
    pbar = jnp.mean(pn, axis=1).T

    g2 = ln_gamma.reshape(1, H)
    b2 = ln_beta.reshape(1, H)
    qpb2 = qp_b.reshape(1, H)

    xbf, wfull = pl.pallas_call(
        _router_body,
        grid=(B // MR,),
        in_specs=[
            pl.BlockSpec((MR, H), lambda i: (i, 0)),
            pl.BlockSpec((MR, H), lambda i: (i, 0)),
            pl.BlockSpec((1, H), lambda i: (0, 0)),
            pl.BlockSpec((1, H), lambda i: (0, 0)),
            pl.BlockSpec((2 * H, H), lambda i: (0, 0)),
            pl.BlockSpec((1, H), lambda i: (0, 0)),
            pl.BlockSpec((H, E), lambda i: (0, 0)),
        ],
        out_specs=[
            pl.BlockSpec((MR, H), lambda i: (i, 0)),
            pl.BlockSpec((MR, E), lambda i: (i, 0)),
        ],
        out_shape=[
            jax.ShapeDtypeStruct((B, H), jnp.bfloat16),
            jax.ShapeDtypeStruct((B, E), f32),
        ],
    )(multimodal_feat, query_feat, g2, b2, qp_W, qpb2, pbar)

    shw1 = sh_W1.astype(jnp.bfloat16)
    shw2 = sh_W2.astype(jnp.bfloat16)
    shared = pl.pallas_call(
        _shared_body,
        grid=(B // MS,),
        in_specs=[
            pl.BlockSpec((MS, H), lambda i: (i, 0)),
            pl.BlockSpec((H, H), lambda i: (0, 0)),
            pl.BlockSpec((1, H), lambda i: (0, 0)),
            pl.BlockSpec((H, P2), lambda i: (0, 0)),
            pl.BlockSpec((1, P2), lambda i: (0, 0)),
        ],
        out_specs=pl.BlockSpec((MS, P2), lambda i: (i, 0)),
        out_shape=jax.ShapeDtypeStruct((B, P2), f32),
    )(xbf, shw1, sh_b1.reshape(1, H), shw2, sh_b2.reshape(1, P2))

    ew1 = exp_W1.astype(jnp.bfloat16)
    ew2 = exp_W2.astype(jnp.bfloat16)
    all_out = pl.pallas_call(
        _experts_body,
        grid=(E, B // ME),
        in_specs=[
            pl.BlockSpec((B, H), lambda e, b: (0, 0)),
            pl.BlockSpec((1, H, H), lambda e, b: (e, 0, 0)),
            pl.BlockSpec((1, 1, H), lambda e, b: (e, 0, 0)),
            pl.BlockSpec((1, H, P2), lambda e, b: (e, 0, 0)),
            pl.BlockSpec((1, 1, P2), lambda e, b: (e, 0, 0)),
        ],
        out_specs=pl.BlockSpec((1, ME, P2), lambda e, b: (e, b, 0)),
        out_shape=jax.ShapeDtypeStruct((E, B, P2), f32),
    )(xbf, ew1, exp_b1.reshape(E, 1, H), ew2, exp_b2.reshape(E, 1, P2))

    out16 = pl.pallas_call(
        _combine_body,
        grid=(B // ME,),
        in_specs=[
            pl.BlockSpec((E, ME, P2), lambda b: (0, b, 0)),
            pl.BlockSpec((ME, E), lambda b: (b, 0)),
            pl.BlockSpec((ME, P2), lambda b: (b, 0)),
            pl.BlockSpec((2 * P2, P2), lambda b: (0, 0)),
            pl.BlockSpec((1, P2), lambda b: (0, 0)),
        ],
        out_specs=pl.BlockSpec((ME, P2), lambda b: (b, 0)),
        out_shape=jax.ShapeDtypeStruct((B, P2), f32),
    )(all_out, wfull, shared, gate_W, gate_b.reshape(1, P2))

    return out16.reshape(B * P, 2)
